# baseline (speedup 1.0000x reference)
import functools

import jax
import jax.numpy as jnp
from jax import lax
from jax.experimental import pallas as pl
from jax.experimental.pallas import tpu as pltpu
from jax.experimental.pallas import tpu_sc as plsc

_BM = 128


def _pack2(lo_bf16, hi_bf16):
    lo = lax.bitcast_convert_type(lo_bf16, jnp.uint16).astype(jnp.uint32)
    hi = lax.bitcast_convert_type(hi_bf16, jnp.uint16).astype(jnp.uint32)
    return lax.bitcast_convert_type(lo | (hi << 16), jnp.int32)


def _unpack2(packed_i32):
    u = lax.bitcast_convert_type(packed_i32, jnp.uint32)
    lo = lax.bitcast_convert_type(u << 16, jnp.float32)
    hi = lax.bitcast_convert_type(u & jnp.uint32(0xFFFF0000), jnp.float32)
    return lo, hi


def _routing(expert_idxs, expert_p, n_experts, bm):
    t, k = expert_idxs.shape
    tk = t * k
    flat_e = expert_idxs.reshape(-1).astype(jnp.int32)
    onehot = (flat_e[:, None] == jnp.arange(n_experts, dtype=jnp.int32)[None, :]
              ).astype(jnp.int32)
    ranks = jnp.cumsum(onehot, axis=0) - onehot
    rank = jnp.take_along_axis(ranks, flat_e[:, None], axis=1)[:, 0]
    counts = jnp.sum(onehot, axis=0)
    blocks_per_e = (counts + bm - 1) // bm
    block_starts = jnp.concatenate(
        [jnp.zeros(1, jnp.int32), jnp.cumsum(blocks_per_e)[:-1].astype(jnp.int32)])
    slots = block_starts[flat_e] * bm + rank
    nb = tk // bm + n_experts
    p = nb * bm
    barange = jnp.arange(nb, dtype=jnp.int32)
    eids = jnp.clip(
        jnp.sum(barange[:, None] >= block_starts[None, :], axis=1) - 1,
        0, n_experts - 1).astype(jnp.int32)
    return slots, eids, p, nb


def _sc_dispatch(x, tok, slots, pf, p):
    t, d = x.shape
    tk = tok.shape[0]
    info = plsc.get_sparse_core_info()
    nc, ns = info.num_cores, info.num_subcores
    nw = nc * ns
    per_w = tk // nw
    ch = min(32, per_w)
    n_ch = per_w // ch
    mesh = plsc.VectorSubcoreMesh(core_axis_name="c", subcore_axis_name="s")

    @functools.partial(
        pl.kernel,
        out_type=(jax.ShapeDtypeStruct((p, d), x.dtype),
                  jax.ShapeDtypeStruct((p, 128), jnp.float32)),
        mesh=mesh,
        scratch_types=[
            pltpu.VMEM((ch,), jnp.int32),
            pltpu.VMEM((ch,), jnp.int32),
            pltpu.VMEM((ch, d), x.dtype),
            pltpu.VMEM((ch, 128), jnp.float32),
            pltpu.SemaphoreType.DMA,
        ],
    )
    def k(x_hbm, tok_hbm, slots_hbm, pf_hbm, xs_hbm, w16_hbm,
          tokv, slotv, rowsv, pfv, sem):
        wid = lax.axis_index("s") * nc + lax.axis_index("c")
        w0 = wid * per_w

        def body(i, carry):
            base = pl.multiple_of(w0 + i * ch, 8)
            pltpu.sync_copy(tok_hbm.at[pl.ds(base, ch)], tokv)
            pltpu.sync_copy(slots_hbm.at[pl.ds(base, ch)], slotv)
            pltpu.sync_copy(pf_hbm.at[pl.ds(base, ch)], pfv)
            pltpu.async_copy(x_hbm.at[tokv], rowsv, sem).wait()
            pltpu.async_copy(rowsv, xs_hbm.at[slotv], sem).wait()
            pltpu.async_copy(pfv, w16_hbm.at[slotv], sem).wait()
            return carry

        lax.fori_loop(0, n_ch, body, 0)

    return k(x, tok, slots, pf)


def _sc_gather(table, idx, m):
    d = table.shape[1]
    info = plsc.get_sparse_core_info()
    nc, ns = info.num_cores, info.num_subcores
    nw = nc * ns
    rows_per_w = m // nw
    ch = min(32, rows_per_w)
    n_ch = rows_per_w // ch
    mesh = plsc.VectorSubcoreMesh(core_axis_name="c", subcore_axis_name="s")

    @functools.partial(
        pl.kernel,
        out_type=jax.ShapeDtypeStruct((m, d), table.dtype),
        mesh=mesh,
        scratch_types=[
            pltpu.VMEM((ch,), jnp.int32),
            pltpu.VMEM((ch, d), table.dtype),
            pltpu.SemaphoreType.DMA,
        ],
    )
    def k(table_hbm, idx_hbm, out_hbm, idxv, rowsv, sem):
        wid = lax.axis_index("s") * nc + lax.axis_index("c")
        w0 = wid * rows_per_w

        def body(i, carry):
            base = pl.multiple_of(w0 + i * ch, 8)
            pltpu.sync_copy(idx_hbm.at[pl.ds(base, ch)], idxv)
            pltpu.async_copy(table_hbm.at[idxv], rowsv, sem).wait()
            pltpu.sync_copy(rowsv, out_hbm.at[pl.ds(base, ch)])
            return carry

        lax.fori_loop(0, n_ch, body, 0)

    return k(table, idx)


def _up_glu(xs, w_up, w16, eids, p, nb, bm):
    e, d, h2 = w_up.shape
    h = h2 // 2
    bn = 1024
    nh = h // bn

    dh = d // 2

    def body(eref, x_ref, wha_ref, whb_ref, wga_ref, wgb_ref, wr_ref, out_ref):
        xa_f, xb_f = _unpack2(x_ref[...])
        xa = xa_f.astype(jnp.bfloat16)
        xb = xb_f.astype(jnp.bfloat16)
        hh = (jnp.dot(xa, wha_ref[0].astype(jnp.bfloat16),
                      preferred_element_type=jnp.float32) +
              jnp.dot(xb, whb_ref[0].astype(jnp.bfloat16),
                      preferred_element_type=jnp.float32))
        g = (jnp.dot(xa, wga_ref[0].astype(jnp.bfloat16),
                     preferred_element_type=jnp.float32) +
             jnp.dot(xb, wgb_ref[0].astype(jnp.bfloat16),
                     preferred_element_type=jnp.float32))
        a = hh * (g * jax.nn.sigmoid(g)) * wr_ref[:, 0:1]
        out_ref[...] = a.astype(jnp.bfloat16)

    gridspec = pltpu.PrefetchScalarGridSpec(
        num_scalar_prefetch=1,
        grid=(nh, nb),
        in_specs=[
            pl.BlockSpec((bm, dh), lambda n, b, er: (b, 0)),
            pl.BlockSpec((1, dh, bn), lambda n, b, er: (er[b], 0, n)),
            pl.BlockSpec((1, dh, bn), lambda n, b, er: (er[b], 1, n)),
            pl.BlockSpec((1, dh, bn), lambda n, b, er: (er[b], 0, nh + n)),
            pl.BlockSpec((1, dh, bn), lambda n, b, er: (er[b], 1, nh + n)),
            pl.BlockSpec((bm, 128), lambda n, b, er: (b, 0)),
        ],
        out_specs=pl.BlockSpec((bm, bn), lambda n, b, er: (b, n)),
    )
    return pl.pallas_call(
        body,
        grid_spec=gridspec,
        out_shape=jax.ShapeDtypeStruct((p, h), jnp.bfloat16),
        compiler_params=pltpu.CompilerParams(
            dimension_semantics=("arbitrary", "arbitrary"),
            vmem_limit_bytes=60 * 1024 * 1024),
    )(eids, xs, w_up, w_up, w_up, w_up, w16)


def _down(a, w_down, eids, p, nb, bm):
    e, h, d = w_down.shape
    bn = 1024
    nd = d // bn

    hh = h // 2

    def body(eref, a_ref, wda_ref, wdb_ref, out_ref):
        a = a_ref[...]
        y = (jnp.dot(a[:, :hh], wda_ref[0].astype(jnp.bfloat16),
                     preferred_element_type=jnp.float32) +
             jnp.dot(a[:, hh:], wdb_ref[0].astype(jnp.bfloat16),
                     preferred_element_type=jnp.float32))
        dhf = y.shape[1] // 2
        out_ref[...] = _pack2(y[:, :dhf].astype(jnp.bfloat16),
                              y[:, dhf:].astype(jnp.bfloat16))

    gridspec = pltpu.PrefetchScalarGridSpec(
        num_scalar_prefetch=1,
        grid=(nd, nb),
        in_specs=[
            pl.BlockSpec((bm, h), lambda n, b, er: (b, 0)),
            pl.BlockSpec((1, hh, bn), lambda n, b, er: (er[b], 0, n)),
            pl.BlockSpec((1, hh, bn), lambda n, b, er: (er[b], 1, n)),
        ],
        out_specs=pl.BlockSpec((bm, bn // 2), lambda n, b, er: (b, n)),
    )
    return pl.pallas_call(
        body,
        grid_spec=gridspec,
        out_shape=jax.ShapeDtypeStruct((p, d // 2), jnp.int32),
        compiler_params=pltpu.CompilerParams(
            dimension_semantics=("arbitrary", "arbitrary"),
            vmem_limit_bytes=60 * 1024 * 1024),
    )(eids, a, w_down, w_down)


def _pairsum(y2, t, k, d):
    bmt = 256

    dp = d // 2
    half = 512

    def body(in_ref, out_ref):
        pieces = []
        for i in range(dp // half):
            lo, hi = _unpack2(in_ref[:, i * half:(i + 1) * half])
            for j in range(1, k):
                lo2, hi2 = _unpack2(
                    in_ref[:, j * dp + i * half:j * dp + (i + 1) * half])
                lo = lo + lo2
                hi = hi + hi2
            pieces += [lo, hi]
        out_ref[...] = jnp.concatenate(pieces, axis=1)

    return pl.pallas_call(
        body,
        grid=(t // bmt,),
        in_specs=[pl.BlockSpec((bmt, k * d // 2), lambda i: (i, 0))],
        out_specs=pl.BlockSpec((bmt, d), lambda i: (i, 0)),
        out_shape=jax.ShapeDtypeStruct((t, d), jnp.float32),
    )(y2)


def kernel(x, expert_p, w_up, w_down, expert_idxs):
    t, d = x.shape
    e = w_up.shape[0]
    k = expert_idxs.shape[1]
    bm = _BM
    slots, eids, p, nb = _routing(expert_idxs, expert_p, e, bm)
    tok = jnp.arange(t * k, dtype=jnp.int32) // k
    pf = jnp.broadcast_to(expert_p.reshape(t * k, 1), (t * k, 128))
    dh = d // 2
    xp = _pack2(x[:, :dh].astype(jnp.bfloat16), x[:, dh:].astype(jnp.bfloat16))
    xs, w16 = _sc_dispatch(xp, tok, slots, pf, p)
    a = _up_glu(xs, w_up, w16, eids, p, nb, bm)
    ysw = _down(a, w_down, eids, p, nb, bm)
    yflat = _sc_gather(ysw, slots, t * k)
    return _pairsum(yflat.reshape(t, k * d // 2), t, k, d)


# speedup vs baseline: 1.0387x; 1.0387x over previous
import functools

import jax
import jax.numpy as jnp
from jax import lax
from jax.experimental import pallas as pl
from jax.experimental.pallas import tpu as pltpu
from jax.experimental.pallas import tpu_sc as plsc

_BM = 256


def _pack2(lo_bf16, hi_bf16):
    lo = lax.bitcast_convert_type(lo_bf16, jnp.uint16).astype(jnp.uint32)
    hi = lax.bitcast_convert_type(hi_bf16, jnp.uint16).astype(jnp.uint32)
    return lax.bitcast_convert_type(lo | (hi << 16), jnp.int32)


def _unpack2(packed_i32):
    u = lax.bitcast_convert_type(packed_i32, jnp.uint32)
    lo = lax.bitcast_convert_type(u << 16, jnp.float32)
    hi = lax.bitcast_convert_type(u & jnp.uint32(0xFFFF0000), jnp.float32)
    return lo, hi


def _routing(expert_idxs, expert_p, n_experts, bm):
    t, k = expert_idxs.shape
    tk = t * k
    flat_e = expert_idxs.reshape(-1).astype(jnp.int32)
    onehot = (flat_e[:, None] == jnp.arange(n_experts, dtype=jnp.int32)[None, :]
              ).astype(jnp.int32)
    ranks = jnp.cumsum(onehot, axis=0) - onehot
    rank = jnp.sum(ranks * onehot, axis=1)
    counts = jnp.sum(onehot, axis=0)
    blocks_per_e = (counts + bm - 1) // bm
    block_starts = jnp.concatenate(
        [jnp.zeros(1, jnp.int32), jnp.cumsum(blocks_per_e)[:-1].astype(jnp.int32)])
    slots = block_starts[flat_e] * bm + rank
    nb = tk // bm + n_experts
    p = nb * bm
    barange = jnp.arange(nb, dtype=jnp.int32)
    eids = jnp.clip(
        jnp.sum(barange[:, None] >= block_starts[None, :], axis=1) - 1,
        0, n_experts - 1).astype(jnp.int32)
    return slots, eids, p, nb


def _sc_dispatch(x, tok, slots, pf, p):
    t, d = x.shape
    tk = tok.shape[0]
    info = plsc.get_sparse_core_info()
    nc, ns = info.num_cores, info.num_subcores
    nw = nc * ns
    per_w = tk // nw
    ch = min(32, per_w)
    n_ch = per_w // ch
    mesh = plsc.VectorSubcoreMesh(core_axis_name="c", subcore_axis_name="s")

    @functools.partial(
        pl.kernel,
        out_type=(jax.ShapeDtypeStruct((p, d), x.dtype),
                  jax.ShapeDtypeStruct((p, 128), jnp.float32)),
        mesh=mesh,
        scratch_types=[
            pltpu.VMEM((ch,), jnp.int32),
            pltpu.VMEM((ch,), jnp.int32),
            pltpu.VMEM((ch, d), x.dtype),
            pltpu.VMEM((ch, 128), jnp.float32),
            pltpu.SemaphoreType.DMA,
        ],
    )
    def k(x_hbm, tok_hbm, slots_hbm, pf_hbm, xs_hbm, w16_hbm,
          tokv, slotv, rowsv, pfv, sem):
        wid = lax.axis_index("s") * nc + lax.axis_index("c")
        w0 = wid * per_w

        def body(i, carry):
            base = pl.multiple_of(w0 + i * ch, 8)
            pltpu.sync_copy(tok_hbm.at[pl.ds(base, ch)], tokv)
            pltpu.sync_copy(slots_hbm.at[pl.ds(base, ch)], slotv)
            pltpu.sync_copy(pf_hbm.at[pl.ds(base, ch)], pfv)
            pltpu.async_copy(x_hbm.at[tokv], rowsv, sem).wait()
            pltpu.async_copy(rowsv, xs_hbm.at[slotv], sem).wait()
            pltpu.async_copy(pfv, w16_hbm.at[slotv], sem).wait()
            return carry

        lax.fori_loop(0, n_ch, body, 0)

    return k(x, tok, slots, pf)


def _sc_gather(table, idx, m):
    d = table.shape[1]
    info = plsc.get_sparse_core_info()
    nc, ns = info.num_cores, info.num_subcores
    nw = nc * ns
    rows_per_w = m // nw
    ch = min(32, rows_per_w)
    n_ch = rows_per_w // ch
    mesh = plsc.VectorSubcoreMesh(core_axis_name="c", subcore_axis_name="s")

    @functools.partial(
        pl.kernel,
        out_type=jax.ShapeDtypeStruct((m, d), table.dtype),
        mesh=mesh,
        scratch_types=[
            pltpu.VMEM((ch,), jnp.int32),
            pltpu.VMEM((ch, d), table.dtype),
            pltpu.SemaphoreType.DMA,
        ],
    )
    def k(table_hbm, idx_hbm, out_hbm, idxv, rowsv, sem):
        wid = lax.axis_index("s") * nc + lax.axis_index("c")
        w0 = wid * rows_per_w

        def body(i, carry):
            base = pl.multiple_of(w0 + i * ch, 8)
            pltpu.sync_copy(idx_hbm.at[pl.ds(base, ch)], idxv)
            pltpu.async_copy(table_hbm.at[idxv], rowsv, sem).wait()
            pltpu.sync_copy(rowsv, out_hbm.at[pl.ds(base, ch)])
            return carry

        lax.fori_loop(0, n_ch, body, 0)

    return k(table, idx)


def _up_glu(xs, w_up, w16, eids, p, nb, bm):
    e, d, h2 = w_up.shape
    h = h2 // 2
    bn = 1024
    nh = h // bn

    dh = d // 2

    def body(eref, x_ref, wha_ref, whb_ref, wga_ref, wgb_ref, wr_ref, out_ref):
        xa_f, xb_f = _unpack2(x_ref[...])
        xa = xa_f.astype(jnp.bfloat16)
        xb = xb_f.astype(jnp.bfloat16)
        hh = (jnp.dot(xa, wha_ref[0].astype(jnp.bfloat16),
                      preferred_element_type=jnp.float32) +
              jnp.dot(xb, whb_ref[0].astype(jnp.bfloat16),
                      preferred_element_type=jnp.float32))
        g = (jnp.dot(xa, wga_ref[0].astype(jnp.bfloat16),
                     preferred_element_type=jnp.float32) +
             jnp.dot(xb, wgb_ref[0].astype(jnp.bfloat16),
                     preferred_element_type=jnp.float32))
        a = hh * (g * jax.nn.sigmoid(g)) * wr_ref[:, 0:1]
        out_ref[...] = a.astype(jnp.bfloat16)

    gridspec = pltpu.PrefetchScalarGridSpec(
        num_scalar_prefetch=1,
        grid=(nh, nb),
        in_specs=[
            pl.BlockSpec((bm, dh), lambda n, b, er: (b, 0)),
            pl.BlockSpec((1, dh, bn), lambda n, b, er: (er[b], 0, n)),
            pl.BlockSpec((1, dh, bn), lambda n, b, er: (er[b], 1, n)),
            pl.BlockSpec((1, dh, bn), lambda n, b, er: (er[b], 0, nh + n)),
            pl.BlockSpec((1, dh, bn), lambda n, b, er: (er[b], 1, nh + n)),
            pl.BlockSpec((bm, 128), lambda n, b, er: (b, 0)),
        ],
        out_specs=pl.BlockSpec((bm, bn), lambda n, b, er: (b, n)),
    )
    return pl.pallas_call(
        body,
        grid_spec=gridspec,
        out_shape=jax.ShapeDtypeStruct((p, h), jnp.bfloat16),
        compiler_params=pltpu.CompilerParams(
            dimension_semantics=("arbitrary", "arbitrary"),
            vmem_limit_bytes=60 * 1024 * 1024),
    )(eids, xs, w_up, w_up, w_up, w_up, w16)


def _down(a, w_down, eids, p, nb, bm):
    e, h, d = w_down.shape
    bn = 1024
    nd = d // bn

    hh = h // 2

    def body(eref, a_ref, wda_ref, wdb_ref, out_ref):
        a = a_ref[...]
        y = (jnp.dot(a[:, :hh], wda_ref[0].astype(jnp.bfloat16),
                     preferred_element_type=jnp.float32) +
             jnp.dot(a[:, hh:], wdb_ref[0].astype(jnp.bfloat16),
                     preferred_element_type=jnp.float32))
        dhf = y.shape[1] // 2
        out_ref[...] = _pack2(y[:, :dhf].astype(jnp.bfloat16),
                              y[:, dhf:].astype(jnp.bfloat16))

    gridspec = pltpu.PrefetchScalarGridSpec(
        num_scalar_prefetch=1,
        grid=(nd, nb),
        in_specs=[
            pl.BlockSpec((bm, h), lambda n, b, er: (b, 0)),
            pl.BlockSpec((1, hh, bn), lambda n, b, er: (er[b], 0, n)),
            pl.BlockSpec((1, hh, bn), lambda n, b, er: (er[b], 1, n)),
        ],
        out_specs=pl.BlockSpec((bm, bn // 2), lambda n, b, er: (b, n)),
    )
    return pl.pallas_call(
        body,
        grid_spec=gridspec,
        out_shape=jax.ShapeDtypeStruct((p, d // 2), jnp.int32),
        compiler_params=pltpu.CompilerParams(
            dimension_semantics=("arbitrary", "arbitrary"),
            vmem_limit_bytes=60 * 1024 * 1024),
    )(eids, a, w_down, w_down)


def _pairsum(y2, t, k, d):
    bmt = 256

    dp = d // 2
    half = 512

    def body(in_ref, out_ref):
        pieces = []
        for i in range(dp // half):
            lo, hi = _unpack2(in_ref[:, i * half:(i + 1) * half])
            for j in range(1, k):
                lo2, hi2 = _unpack2(
                    in_ref[:, j * dp + i * half:j * dp + (i + 1) * half])
                lo = lo + lo2
                hi = hi + hi2
            pieces += [lo, hi]
        out_ref[...] = jnp.concatenate(pieces, axis=1)

    return pl.pallas_call(
        body,
        grid=(t // bmt,),
        in_specs=[pl.BlockSpec((bmt, k * d // 2), lambda i: (i, 0))],
        out_specs=pl.BlockSpec((bmt, d), lambda i: (i, 0)),
        out_shape=jax.ShapeDtypeStruct((t, d), jnp.float32),
    )(y2)


def kernel(x, expert_p, w_up, w_down, expert_idxs):
    t, d = x.shape
    e = w_up.shape[0]
    k = expert_idxs.shape[1]
    bm = _BM
    slots, eids, p, nb = _routing(expert_idxs, expert_p, e, bm)
    tok = jnp.arange(t * k, dtype=jnp.int32) // k
    pf = jnp.broadcast_to(expert_p.reshape(t * k, 1), (t * k, 128))
    dh = d // 2
    xp = _pack2(x[:, :dh].astype(jnp.bfloat16), x[:, dh:].astype(jnp.bfloat16))
    xs, w16 = _sc_dispatch(xp, tok, slots, pf, p)
    a = _up_glu(xs, w_up, w16, eids, p, nb, bm)
    ysw = _down(a, w_down, eids, p, nb, bm)
    yflat = _sc_gather(ysw, slots, t * k)
    return _pairsum(yflat.reshape(t, k * d // 2), t, k, d)
